# baseline (speedup 1.0000x reference)
import functools

import jax
import jax.numpy as jnp
from jax import lax
from jax.experimental import pallas as pl
from jax.experimental.pallas import tpu as pltpu
from jax.experimental.pallas import tpu_sc as plsc

B = 32
N = 12288
NKC = 9
K = 10
L = 16
CH = 3072
NCHUNK = N // CH
GRP = CH // L
WG = 32
PEN = 1e20


def _sc_topk_candidates(kpts_t, cpt_t, pcld_t, seg_t):
    info = plsc.get_sparse_core_info()
    nc = info.num_cores
    mesh = plsc.VectorSubcoreMesh(core_axis_name="c", subcore_axis_name="s")

    @functools.partial(
        pl.kernel,
        mesh=mesh,
        compiler_params=pltpu.CompilerParams(needs_layout_passes=False),
        out_type=jax.ShapeDtypeStruct((B, NKC, 3 * L), jnp.float32),
        scratch_types=[
            pltpu.VMEM((24, CH), jnp.float32),
            pltpu.VMEM((3, CH), jnp.float32),
            pltpu.VMEM((3, CH), jnp.float32),
            pltpu.VMEM((2, CH), jnp.float32),
            pltpu.VMEM((NKC, L), jnp.float32),
            pltpu.VMEM((NKC, L), jnp.float32),
            pltpu.VMEM((NKC, L), jnp.float32),
            pltpu.VMEM((NKC, L), jnp.float32),
            pltpu.VMEM((NKC, L), jnp.float32),
            pltpu.VMEM((NKC, 3 * L), jnp.float32),
            pltpu.VMEM((4, L), jnp.float32),
            pltpu.SMEM((NKC * WG,), jnp.int32),
            pltpu.SMEM((NKC,), jnp.int32),
            pltpu.SemaphoreType.DMA,
        ],
    )
    def topk_kernel(kpts_hbm, cpt_hbm, pcld_hbm, seg_hbm, out_hbm,
                    kbuf, cbuf, pbuf, sbuf, bv, bcx, bcy, bcz, thr, obuf,
                    prm, que, qn, sem):
        b = lax.axis_index("s") * nc + lax.axis_index("c")
        iota = lax.iota(jnp.int32, L)
        inf16 = jnp.full((L,), jnp.inf, jnp.float32)
        zero16 = jnp.zeros((L,), jnp.float32)
        iz = jnp.zeros((L,), jnp.int32)
        i1 = jnp.full((L,), 1, jnp.int32)
        i2 = jnp.full((L,), 2, jnp.int32)

        for j in range(NKC):
            bv[j] = inf16
            thr[j] = inf16
            bcx[j] = zero16
            bcy[j] = zero16
            bcz[j] = zero16
            qn[j] = 0

        def merge_group(j, g):
            gl = g * L
            s0 = sbuf[0, pl.ds(gl, L)]
            s1 = sbuf[1, pl.ds(gl, L)]
            pen = jnp.where(s1 > s0, 0.0, PEN).astype(jnp.float32)
            if j < 8:
                x = kbuf[3 * j, pl.ds(gl, L)]
                y = kbuf[3 * j + 1, pl.ds(gl, L)]
                z = kbuf[3 * j + 2, pl.ds(gl, L)]
            else:
                x = cbuf[0, pl.ds(gl, L)]
                y = cbuf[1, pl.ds(gl, L)]
                z = cbuf[2, pl.ds(gl, L)]
            kk = x * x + y * y + z * z + pen
            admit = kk < thr[j]
            keyv = jnp.where(admit, kk, jnp.inf)
            px = pbuf[0, pl.ds(gl, L)]
            py = pbuf[1, pl.ds(gl, L)]
            pz = pbuf[2, pl.ds(gl, L)]
            prm[0] = px + x
            prm[1] = py + y
            prm[2] = pz + z
            sk, sl = plsc.sort_key_val(keyv, iota)
            scx = plsc.load_gather(prm, [iz, sl])
            scy = plsc.load_gather(prm, [i1, sl])
            scz = plsc.load_gather(prm, [i2, sl])
            obv = bv[j]
            take = sk <= obv
            mv = jnp.minimum(sk, obv)
            prm[0] = jnp.where(take, scx, bcx[j])
            prm[1] = jnp.where(take, scy, bcy[j])
            prm[2] = jnp.where(take, scz, bcz[j])
            nbv, ml = plsc.sort_key_val(mv, iota, descending=True)
            prm[3] = nbv
            bcx[j] = plsc.load_gather(prm, [iz, ml])
            bcy[j] = plsc.load_gather(prm, [i1, ml])
            bcz[j] = plsc.load_gather(prm, [i2, ml])
            bv[j] = nbv
            thr[j] = plsc.load_gather(
                prm, [jnp.full((L,), 3, jnp.int32),
                      jnp.full((L,), 6, jnp.int32)])

        def chunk_body(ch, carry):
            off = ch * CH
            c1 = pltpu.async_copy(
                kpts_hbm.at[b, :, pl.ds(off, CH)], kbuf, sem)
            c2 = pltpu.async_copy(
                cpt_hbm.at[b, :, pl.ds(off, CH)], cbuf, sem)
            c3 = pltpu.async_copy(
                pcld_hbm.at[b, :, pl.ds(off, CH)], pbuf, sem)
            c4 = pltpu.async_copy(
                seg_hbm.at[b, :, pl.ds(off, CH)], sbuf, sem)
            c1.wait()
            c2.wait()
            c3.wait()
            c4.wait()

            def window_body(w, carry2):
                def group_body(gg, carry3):
                    g = w * WG + gg
                    gl = g * L
                    s0 = sbuf[0, pl.ds(gl, L)]
                    s1 = sbuf[1, pl.ds(gl, L)]
                    pen = jnp.where(s1 > s0, 0.0, PEN).astype(jnp.float32)
                    for j in range(NKC):
                        if j < 8:
                            x = kbuf[3 * j, pl.ds(gl, L)]
                            y = kbuf[3 * j + 1, pl.ds(gl, L)]
                            z = kbuf[3 * j + 2, pl.ds(gl, L)]
                        else:
                            x = cbuf[0, pl.ds(gl, L)]
                            y = cbuf[1, pl.ds(gl, L)]
                            z = cbuf[2, pl.ds(gl, L)]
                        kk = x * x + y * y + z * z + pen
                        adm = jnp.any(kk < thr[j]).astype(jnp.int32)
                        c = qn[j]
                        que[j * WG + c] = g
                        qn[j] = c + adm
                    return carry3

                lax.fori_loop(0, WG, group_body, 0)

                for j in range(NKC):
                    n = qn[j]

                    def drain_body(i, j=j):
                        merge_group(j, que[j * WG + i])
                        return i + 1

                    lax.while_loop(lambda i, n=n: i < n, drain_body, 0)
                    qn[j] = 0
                return carry2

            lax.fori_loop(0, GRP // WG, window_body, 0)
            return carry

        lax.fori_loop(0, NCHUNK, chunk_body, 0)

        for j in range(NKC):
            obuf[j, pl.ds(0, L)] = bcx[j]
            obuf[j, pl.ds(L, L)] = bcy[j]
            obuf[j, pl.ds(2 * L, L)] = bcz[j]
        pltpu.sync_copy(obuf, out_hbm.at[b])

    return topk_kernel(kpts_t, cpt_t, pcld_t, seg_t)


def _tc_cluster_kabsch(cands, mesh_kpts):
    def body(cand_ref, mesh_ref, r_ref, t_ref, v_ref):
        c = cand_ref[...]
        comps = [c[:, :, 6:16], c[:, :, 22:32], c[:, :, 38:48]]
        voted = []
        for v in comps:
            mean = jnp.mean(v, axis=-1, keepdims=True)
            d = v - mean
            std = jnp.sqrt(jnp.mean(d * d, axis=-1, keepdims=True))
            m = jnp.logical_and(v >= mean - std, v <= mean + std)
            m = m.astype(jnp.float32)
            voted.append(jnp.sum(v * m, axis=-1) / (jnp.sum(m, axis=-1) + 1e-8))
        vx, vy, vz = voted
        mk = mesh_ref[...]
        A = [mk[:, :, 0], mk[:, :, 1], mk[:, :, 2]]
        Bc = [vx, vy, vz]
        cA = [jnp.mean(a, axis=1) for a in A]
        cB = [jnp.mean(bb, axis=1) for bb in Bc]
        Am = [a - ca[:, None] for a, ca in zip(A, cA)]
        Bm = [bb - cb[:, None] for bb, cb in zip(Bc, cB)]
        H = [[jnp.sum(Am[d] * Bm[e], axis=1) for e in range(3)]
             for d in range(3)]
        S = [[H[0][d] * H[0][e] + H[1][d] * H[1][e] + H[2][d] * H[2][e]
              for e in range(3)] for d in range(3)]
        one = jnp.ones_like(S[0][0])
        V = [[one * (1.0 if d == e else 0.0) for e in range(3)] for d in range(3)]

        def jacobi(S, V, p, q):
            app, aqq, apq = S[p][p], S[q][q], S[p][q]
            small = jnp.abs(apq) < 1e-30
            theta = (aqq - app) / (2.0 * jnp.where(small, 1.0, apq))
            t = jnp.sign(theta) / (jnp.abs(theta) + jnp.sqrt(theta * theta + 1.0))
            t = jnp.where(small, 0.0, t)
            cth = 1.0 / jnp.sqrt(t * t + 1.0)
            sth = t * cth
            r = 3 - p - q
            Srp, Srq = S[r][p], S[r][q]
            nS = [row[:] for row in S]
            nS[p][p] = app - t * apq
            nS[q][q] = aqq + t * apq
            nS[p][q] = jnp.zeros_like(apq)
            nS[q][p] = nS[p][q]
            nS[r][p] = cth * Srp - sth * Srq
            nS[p][r] = nS[r][p]
            nS[r][q] = sth * Srp + cth * Srq
            nS[q][r] = nS[r][q]
            nV = [row[:] for row in V]
            for rr in range(3):
                vp, vq = V[rr][p], V[rr][q]
                nV[rr][p] = cth * vp - sth * vq
                nV[rr][q] = sth * vp + cth * vq
            return nS, nV

        for _ in range(10):
            for (p, q) in ((0, 1), (0, 2), (1, 2)):
                S, V = jacobi(S, V, p, q)

        lam = [S[0][0], S[1][1], S[2][2]]

        def eig_swap(lam, V, i, j):
            do = lam[i] < lam[j]
            li, lj = lam[i], lam[j]
            lam = lam[:]
            lam[i] = jnp.where(do, lj, li)
            lam[j] = jnp.where(do, li, lj)
            nV = [row[:] for row in V]
            for rr in range(3):
                vi, vj = V[rr][i], V[rr][j]
                nV[rr][i] = jnp.where(do, vj, vi)
                nV[rr][j] = jnp.where(do, vi, vj)
            return lam, nV

        for (i, j) in ((0, 1), (0, 2), (1, 2)):
            lam, V = eig_swap(lam, V, i, j)

        sig = [jnp.sqrt(jnp.maximum(l, 0.0)) for l in lam]
        v1 = [V[d][0] for d in range(3)]
        v2 = [V[d][1] for d in range(3)]
        is1 = 1.0 / jnp.maximum(sig[0], 1e-30)
        is2 = 1.0 / jnp.maximum(sig[1], 1e-30)
        u1 = [sum(H[d][k] * v1[k] for k in range(3)) * is1 for d in range(3)]
        u2 = [sum(H[d][k] * v2[k] for k in range(3)) * is2 for d in range(3)]

        def cross(a, b):
            return [a[1] * b[2] - a[2] * b[1],
                    a[2] * b[0] - a[0] * b[2],
                    a[0] * b[1] - a[1] * b[0]]

        v3 = cross(v1, v2)
        u3 = cross(u1, u2)
        R = [[v1[d] * u1[e] + v2[d] * u2[e] + v3[d] * u3[e] for e in range(3)]
             for d in range(3)]
        for _ in range(2):
            G = [[sum(R[k][d] * R[k][e] for k in range(3)) for e in range(3)]
                 for d in range(3)]
            W = [[(3.0 * (1.0 if d == e else 0.0) - G[d][e]) * 0.5
                  for e in range(3)] for d in range(3)]
            R = [[sum(R[d][k] * W[k][e] for k in range(3)) for e in range(3)]
                 for d in range(3)]
        tvec = [cB[d] - sum(R[d][k] * cA[k] for k in range(3)) for d in range(3)]

        r_ref[...] = jnp.stack(
            [jnp.stack([R[d][e] for e in range(3)], axis=-1) for d in range(3)],
            axis=1)
        t_ref[...] = jnp.stack(tvec, axis=-1)
        v_ref[...] = jnp.stack([vx, vy, vz], axis=-1)

    return pl.pallas_call(
        body,
        out_shape=(
            jax.ShapeDtypeStruct((B, 3, 3), jnp.float32),
            jax.ShapeDtypeStruct((B, 3), jnp.float32),
            jax.ShapeDtypeStruct((B, NKC, 3), jnp.float32),
        ),
    )(cands, mesh_kpts)


def kernel(pcld_input, kpts_pre_input, cpt_pre_input, seg_pre_input,
           mesh_kpts_input):
    kpts_t = jnp.transpose(kpts_pre_input.reshape(B, N, 24), (0, 2, 1))
    cpt_t = jnp.transpose(cpt_pre_input.reshape(B, N, 3), (0, 2, 1))
    pcld_t = jnp.transpose(pcld_input, (0, 2, 1))
    seg_t = jnp.transpose(seg_pre_input, (0, 2, 1))
    cands = _sc_topk_candidates(kpts_t, cpt_t, pcld_t, seg_t)
    batch_R, batch_t, kpts_voted = _tc_cluster_kabsch(cands, mesh_kpts_input)
    return (batch_R, batch_t, kpts_voted)


# speedup vs baseline: 1.0410x; 1.0410x over previous
import functools

import jax
import jax.numpy as jnp
from jax import lax
from jax.experimental import pallas as pl
from jax.experimental.pallas import tpu as pltpu
from jax.experimental.pallas import tpu_sc as plsc

B = 32
N = 12288
NKC = 9
K = 10
L = 16
CH = 3072
NCHUNK = N // CH
GRP = CH // L
WG = 64
PEN = 1e20


def _sc_topk_candidates(kpts_t, cpt_t, pcld_t, seg_t):
    info = plsc.get_sparse_core_info()
    nc = info.num_cores
    mesh = plsc.VectorSubcoreMesh(core_axis_name="c", subcore_axis_name="s")

    @functools.partial(
        pl.kernel,
        mesh=mesh,
        compiler_params=pltpu.CompilerParams(needs_layout_passes=False),
        out_type=jax.ShapeDtypeStruct((B, NKC, 3 * L), jnp.float32),
        scratch_types=[
            pltpu.VMEM((24, CH), jnp.float32),
            pltpu.VMEM((3, CH), jnp.float32),
            pltpu.VMEM((3, CH), jnp.float32),
            pltpu.VMEM((2, CH), jnp.float32),
            pltpu.VMEM((NKC, L), jnp.float32),
            pltpu.VMEM((NKC, L), jnp.float32),
            pltpu.VMEM((NKC, L), jnp.float32),
            pltpu.VMEM((NKC, L), jnp.float32),
            pltpu.VMEM((NKC, L), jnp.float32),
            pltpu.VMEM((NKC, 3 * L), jnp.float32),
            pltpu.VMEM((4, L), jnp.float32),
            pltpu.SMEM((NKC * WG,), jnp.int32),
            pltpu.SMEM((NKC,), jnp.int32),
            pltpu.SemaphoreType.DMA,
        ],
    )
    def topk_kernel(kpts_hbm, cpt_hbm, pcld_hbm, seg_hbm, out_hbm,
                    kbuf, cbuf, pbuf, sbuf, bv, bcx, bcy, bcz, thr, obuf,
                    prm, que, qn, sem):
        b = lax.axis_index("s") * nc + lax.axis_index("c")
        iota = lax.iota(jnp.int32, L)
        inf16 = jnp.full((L,), jnp.inf, jnp.float32)
        zero16 = jnp.zeros((L,), jnp.float32)
        iz = jnp.zeros((L,), jnp.int32)
        i1 = jnp.full((L,), 1, jnp.int32)
        i2 = jnp.full((L,), 2, jnp.int32)

        for j in range(NKC):
            bv[j] = inf16
            thr[j] = inf16
            bcx[j] = zero16
            bcy[j] = zero16
            bcz[j] = zero16
            qn[j] = 0

        def merge_group(j, g):
            gl = g * L
            s0 = sbuf[0, pl.ds(gl, L)]
            s1 = sbuf[1, pl.ds(gl, L)]
            pen = jnp.where(s1 > s0, 0.0, PEN).astype(jnp.float32)
            if j < 8:
                x = kbuf[3 * j, pl.ds(gl, L)]
                y = kbuf[3 * j + 1, pl.ds(gl, L)]
                z = kbuf[3 * j + 2, pl.ds(gl, L)]
            else:
                x = cbuf[0, pl.ds(gl, L)]
                y = cbuf[1, pl.ds(gl, L)]
                z = cbuf[2, pl.ds(gl, L)]
            kk = x * x + y * y + z * z + pen
            admit = kk < thr[j]
            keyv = jnp.where(admit, kk, jnp.inf)
            px = pbuf[0, pl.ds(gl, L)]
            py = pbuf[1, pl.ds(gl, L)]
            pz = pbuf[2, pl.ds(gl, L)]
            prm[0] = px + x
            prm[1] = py + y
            prm[2] = pz + z
            sk, sl = plsc.sort_key_val(keyv, iota)
            scx = plsc.load_gather(prm, [iz, sl])
            scy = plsc.load_gather(prm, [i1, sl])
            scz = plsc.load_gather(prm, [i2, sl])
            obv = bv[j]
            take = sk <= obv
            mv = jnp.minimum(sk, obv)
            prm[0] = jnp.where(take, scx, bcx[j])
            prm[1] = jnp.where(take, scy, bcy[j])
            prm[2] = jnp.where(take, scz, bcz[j])
            nbv, ml = plsc.sort_key_val(mv, iota, descending=True)
            prm[3] = nbv
            bcx[j] = plsc.load_gather(prm, [iz, ml])
            bcy[j] = plsc.load_gather(prm, [i1, ml])
            bcz[j] = plsc.load_gather(prm, [i2, ml])
            bv[j] = nbv
            thr[j] = plsc.load_gather(
                prm, [jnp.full((L,), 3, jnp.int32),
                      jnp.full((L,), 6, jnp.int32)])

        def chunk_body(ch, carry):
            off = ch * CH
            c1 = pltpu.async_copy(
                kpts_hbm.at[b, :, pl.ds(off, CH)], kbuf, sem)
            c2 = pltpu.async_copy(
                cpt_hbm.at[b, :, pl.ds(off, CH)], cbuf, sem)
            c3 = pltpu.async_copy(
                pcld_hbm.at[b, :, pl.ds(off, CH)], pbuf, sem)
            c4 = pltpu.async_copy(
                seg_hbm.at[b, :, pl.ds(off, CH)], sbuf, sem)
            c1.wait()
            c2.wait()
            c3.wait()
            c4.wait()

            def window_body(w, carry2):
                def group_body(gg, carry3):
                    g = w * WG + gg
                    gl = g * L
                    s0 = sbuf[0, pl.ds(gl, L)]
                    s1 = sbuf[1, pl.ds(gl, L)]
                    pen = jnp.where(s1 > s0, 0.0, PEN).astype(jnp.float32)
                    for j in range(NKC):
                        if j < 8:
                            x = kbuf[3 * j, pl.ds(gl, L)]
                            y = kbuf[3 * j + 1, pl.ds(gl, L)]
                            z = kbuf[3 * j + 2, pl.ds(gl, L)]
                        else:
                            x = cbuf[0, pl.ds(gl, L)]
                            y = cbuf[1, pl.ds(gl, L)]
                            z = cbuf[2, pl.ds(gl, L)]
                        kk = x * x + y * y + z * z + pen
                        adm = jnp.any(kk < thr[j]).astype(jnp.int32)
                        c = qn[j]
                        que[j * WG + c] = g
                        qn[j] = c + adm
                    return carry3

                lax.fori_loop(0, WG, group_body, 0)

                for j in range(NKC):
                    n = qn[j]

                    def drain_body(i, j=j):
                        merge_group(j, que[j * WG + i])
                        return i + 1

                    lax.while_loop(lambda i, n=n: i < n, drain_body, 0)
                    qn[j] = 0
                return carry2

            lax.fori_loop(0, GRP // WG, window_body, 0)
            return carry

        lax.fori_loop(0, NCHUNK, chunk_body, 0)

        for j in range(NKC):
            obuf[j, pl.ds(0, L)] = bcx[j]
            obuf[j, pl.ds(L, L)] = bcy[j]
            obuf[j, pl.ds(2 * L, L)] = bcz[j]
        pltpu.sync_copy(obuf, out_hbm.at[b])

    return topk_kernel(kpts_t, cpt_t, pcld_t, seg_t)


def _tc_cluster_kabsch(cands, mesh_kpts):
    def body(cand_ref, mesh_ref, r_ref, t_ref, v_ref):
        c = cand_ref[...]
        comps = [c[:, :, 6:16], c[:, :, 22:32], c[:, :, 38:48]]
        voted = []
        for v in comps:
            mean = jnp.mean(v, axis=-1, keepdims=True)
            d = v - mean
            std = jnp.sqrt(jnp.mean(d * d, axis=-1, keepdims=True))
            m = jnp.logical_and(v >= mean - std, v <= mean + std)
            m = m.astype(jnp.float32)
            voted.append(jnp.sum(v * m, axis=-1) / (jnp.sum(m, axis=-1) + 1e-8))
        vx, vy, vz = voted
        mk = mesh_ref[...]
        A = [mk[:, :, 0], mk[:, :, 1], mk[:, :, 2]]
        Bc = [vx, vy, vz]
        cA = [jnp.mean(a, axis=1) for a in A]
        cB = [jnp.mean(bb, axis=1) for bb in Bc]
        Am = [a - ca[:, None] for a, ca in zip(A, cA)]
        Bm = [bb - cb[:, None] for bb, cb in zip(Bc, cB)]
        H = [[jnp.sum(Am[d] * Bm[e], axis=1) for e in range(3)]
             for d in range(3)]
        S = [[H[0][d] * H[0][e] + H[1][d] * H[1][e] + H[2][d] * H[2][e]
              for e in range(3)] for d in range(3)]
        one = jnp.ones_like(S[0][0])
        V = [[one * (1.0 if d == e else 0.0) for e in range(3)] for d in range(3)]

        def jacobi(S, V, p, q):
            app, aqq, apq = S[p][p], S[q][q], S[p][q]
            small = jnp.abs(apq) < 1e-30
            theta = (aqq - app) / (2.0 * jnp.where(small, 1.0, apq))
            t = jnp.sign(theta) / (jnp.abs(theta) + jnp.sqrt(theta * theta + 1.0))
            t = jnp.where(small, 0.0, t)
            cth = 1.0 / jnp.sqrt(t * t + 1.0)
            sth = t * cth
            r = 3 - p - q
            Srp, Srq = S[r][p], S[r][q]
            nS = [row[:] for row in S]
            nS[p][p] = app - t * apq
            nS[q][q] = aqq + t * apq
            nS[p][q] = jnp.zeros_like(apq)
            nS[q][p] = nS[p][q]
            nS[r][p] = cth * Srp - sth * Srq
            nS[p][r] = nS[r][p]
            nS[r][q] = sth * Srp + cth * Srq
            nS[q][r] = nS[r][q]
            nV = [row[:] for row in V]
            for rr in range(3):
                vp, vq = V[rr][p], V[rr][q]
                nV[rr][p] = cth * vp - sth * vq
                nV[rr][q] = sth * vp + cth * vq
            return nS, nV

        for _ in range(10):
            for (p, q) in ((0, 1), (0, 2), (1, 2)):
                S, V = jacobi(S, V, p, q)

        lam = [S[0][0], S[1][1], S[2][2]]

        def eig_swap(lam, V, i, j):
            do = lam[i] < lam[j]
            li, lj = lam[i], lam[j]
            lam = lam[:]
            lam[i] = jnp.where(do, lj, li)
            lam[j] = jnp.where(do, li, lj)
            nV = [row[:] for row in V]
            for rr in range(3):
                vi, vj = V[rr][i], V[rr][j]
                nV[rr][i] = jnp.where(do, vj, vi)
                nV[rr][j] = jnp.where(do, vi, vj)
            return lam, nV

        for (i, j) in ((0, 1), (0, 2), (1, 2)):
            lam, V = eig_swap(lam, V, i, j)

        sig = [jnp.sqrt(jnp.maximum(l, 0.0)) for l in lam]
        v1 = [V[d][0] for d in range(3)]
        v2 = [V[d][1] for d in range(3)]
        is1 = 1.0 / jnp.maximum(sig[0], 1e-30)
        is2 = 1.0 / jnp.maximum(sig[1], 1e-30)
        u1 = [sum(H[d][k] * v1[k] for k in range(3)) * is1 for d in range(3)]
        u2 = [sum(H[d][k] * v2[k] for k in range(3)) * is2 for d in range(3)]

        def cross(a, b):
            return [a[1] * b[2] - a[2] * b[1],
                    a[2] * b[0] - a[0] * b[2],
                    a[0] * b[1] - a[1] * b[0]]

        v3 = cross(v1, v2)
        u3 = cross(u1, u2)
        R = [[v1[d] * u1[e] + v2[d] * u2[e] + v3[d] * u3[e] for e in range(3)]
             for d in range(3)]
        for _ in range(2):
            G = [[sum(R[k][d] * R[k][e] for k in range(3)) for e in range(3)]
                 for d in range(3)]
            W = [[(3.0 * (1.0 if d == e else 0.0) - G[d][e]) * 0.5
                  for e in range(3)] for d in range(3)]
            R = [[sum(R[d][k] * W[k][e] for k in range(3)) for e in range(3)]
                 for d in range(3)]
        tvec = [cB[d] - sum(R[d][k] * cA[k] for k in range(3)) for d in range(3)]

        r_ref[...] = jnp.stack(
            [jnp.stack([R[d][e] for e in range(3)], axis=-1) for d in range(3)],
            axis=1)
        t_ref[...] = jnp.stack(tvec, axis=-1)
        v_ref[...] = jnp.stack([vx, vy, vz], axis=-1)

    return pl.pallas_call(
        body,
        out_shape=(
            jax.ShapeDtypeStruct((B, 3, 3), jnp.float32),
            jax.ShapeDtypeStruct((B, 3), jnp.float32),
            jax.ShapeDtypeStruct((B, NKC, 3), jnp.float32),
        ),
    )(cands, mesh_kpts)


def kernel(pcld_input, kpts_pre_input, cpt_pre_input, seg_pre_input,
           mesh_kpts_input):
    kpts_t = jnp.transpose(kpts_pre_input.reshape(B, N, 24), (0, 2, 1))
    cpt_t = jnp.transpose(cpt_pre_input.reshape(B, N, 3), (0, 2, 1))
    pcld_t = jnp.transpose(pcld_input, (0, 2, 1))
    seg_t = jnp.transpose(seg_pre_input, (0, 2, 1))
    cands = _sc_topk_candidates(kpts_t, cpt_t, pcld_t, seg_t)
    batch_R, batch_t, kpts_voted = _tc_cluster_kabsch(cands, mesh_kpts_input)
    return (batch_R, batch_t, kpts_voted)
